# baseline (speedup 1.0000x reference)
import functools

import jax
import jax.numpy as jnp
from jax import lax
from jax.experimental import pallas as pl
from jax.experimental.pallas import tpu as pltpu
from jax.experimental.pallas import tpu_sc as plsc

N = 10000
D = 128
DH = D // 2
E = 320000

NC = 2
NS = 16
EPS = E // NS
CHUNK = 80
NCHUNKS = EPS // CHUNK
IDXB = 25
NBLK = NCHUNKS // IDXB
CROWS = E // CHUNK
CW = 8
NP = 10240
RPT = NP // NS

_F32 = jnp.float32


NBUF = 5


def _sc_agg_body(with_counts, y_hbm, e2_hbm, *refs):
    if with_counts:
        (out_s, out_c, src_i, dst_i, r0, r1, r2, r3, r4, ones_v, zbuf, zcnt,
         agg_sh, cnt_sh, b0, b1, b2_, b3, b4,
         isem0, isem1, csem) = refs
    else:
        (out_s, src_i, dst_i, r0, r1, r2, r3, r4, zbuf,
         agg_sh, b0, b1, b2_, b3, b4,
         isem0, isem1) = refs
    rows = (r0, r1, r2, r3, r4)
    bsem = (b0, b1, b2_, b3, b4)
    isem = (isem0, isem1)

    cid = lax.axis_index("c")
    sid = lax.axis_index("s")

    base = sid * RPT

    crow = sid * NCHUNKS

    def _fire_idx(r, p):
        row = crow + r * IDXB
        pltpu.async_copy(e2_hbm.at[0, pl.ds(row, IDXB)], src_i.at[p], isem[p])
        pltpu.async_copy(e2_hbm.at[1, pl.ds(row, IDXB)], dst_i.at[p], isem[p])

    def _wait_idx(p):
        pltpu.make_async_copy(e2_hbm.at[0, pl.ds(0, IDXB)], src_i.at[p],
                              isem[p]).wait()
        pltpu.make_async_copy(e2_hbm.at[0, pl.ds(0, IDXB)], dst_i.at[p],
                              isem[p]).wait()

    def _add_off(p):
        for c in range(IDXB):
            for k in range(CHUNK // 16):
                sl = pl.ds(k * 16, 16)
                src_i[p, c, sl] = src_i[p, c, sl] * 2 + cid

    def _fire_gather(p, u, b):
        pltpu.async_copy(y_hbm.at[src_i.at[p, u]], rows[b], bsem[b])

    def _wait_gather(b):
        pltpu.make_async_copy(y_hbm.at[pl.ds(0, CHUNK)], rows[b],
                              bsem[b]).wait()

    def _fire_scat(p, u, b):
        pltpu.async_copy(rows[b], agg_sh.at[dst_i.at[p, u]], bsem[b],
                         add=True)

    _wait_scat = _wait_gather

    def _fire_cnt(p, u):
        pltpu.async_copy(ones_v, cnt_sh.at[dst_i.at[p, u]], csem, add=True)

    def _wait_cnt():
        pltpu.make_async_copy(y_hbm.at[pl.ds(0, (CHUNK * CW) // DH)], ones_v,
                              csem).wait()

    _fire_idx(0, 0)
    _fire_idx(1, 1)

    def _zrow(i, _):
        for c in range(DH // 16):
            zbuf[i, pl.ds(c * 16, 16)] = jnp.zeros((16,), _F32)
        if with_counts:
            zcnt[i, pl.ds(0, 16)] = jnp.zeros((16,), _F32)
        return 0

    lax.fori_loop(0, RPT, _zrow, 0)

    if with_counts:
        def _orow(i, _):
            ones_v[i, pl.ds(0, 16)] = jnp.ones((16,), _F32)
            return 0
        lax.fori_loop(0, CHUNK, _orow, 0)

    pltpu.sync_copy(zbuf, agg_sh.at[pl.ds(base, RPT)])
    if with_counts:
        pltpu.sync_copy(zcnt, cnt_sh.at[pl.ds(base, RPT)])

    _wait_idx(0)
    _add_off(0)
    _fire_gather(0, 0, 0)
    _fire_gather(0, 1, 1)

    plsc.subcore_barrier()

    def _outer(t, _):
        for p in range(2):
            for u in range(IDXB):
                b = u % NBUF
                _wait_gather(b)
                _fire_scat(p, u, b)
                if with_counts:
                    @pl.when(cid == (u % 2))
                    def _():
                        if p == 0 and u < 2:
                            @pl.when(t > 0)
                            def _():
                                _wait_cnt()
                        else:
                            _wait_cnt()
                        _fire_cnt(p, u)
                un = u + 2
                if un < IDXB:
                    b2 = un % NBUF
                    if p == 0 and u < 3:
                        @pl.when(t > 0)
                        def _():
                            _wait_scat(b2)
                    else:
                        _wait_scat(b2)
                    _fire_gather(p, un, b2)
                    if u == 2:
                        if p == 0:
                            @pl.when(t > 0)
                            def _():
                                _fire_idx(2 * t + 1, 1)
                        else:
                            @pl.when(t < NBLK // 2 - 1)
                            def _():
                                _fire_idx(2 * t + 2, 0)
                else:
                    uo = un - IDXB
                    if p == 0:
                        if u == IDXB - 2:
                            _wait_idx(1)
                            _add_off(1)
                        _wait_scat(uo)
                        _fire_gather(1, uo, uo)
                    else:
                        @pl.when(t < NBLK // 2 - 1)
                        def _():
                            if u == IDXB - 2:
                                _wait_idx(0)
                                _add_off(0)
                            _wait_scat(uo)
                            _fire_gather(0, uo, uo)
        return 0

    lax.fori_loop(0, NBLK // 2, _outer, 0)

    for b in range(NBUF):
        _wait_scat(b)
    if with_counts:
        _wait_cnt()

    plsc.subcore_barrier()

    pltpu.sync_copy(agg_sh.at[pl.ds(base, RPT)],
                    out_s.at[pl.ds(base, RPT), pl.ds(cid * DH, DH)])
    if with_counts:
        pltpu.sync_copy(cnt_sh.at[pl.ds(base, RPT)],
                        out_c.at[cid, pl.ds(base, RPT)])


def _make_sc_agg(with_counts):
    if with_counts:
        out_type = (jax.ShapeDtypeStruct((NP, D), _F32),
                    jax.ShapeDtypeStruct((NC, NP, CW), _F32))
        scratch = (
            [pltpu.VMEM((2, IDXB, CHUNK), jnp.int32),
             pltpu.VMEM((2, IDXB, CHUNK), jnp.int32)]
            + [pltpu.VMEM((CHUNK, DH), _F32)] * NBUF
            + [pltpu.VMEM((CHUNK, CW), _F32),
               pltpu.VMEM((RPT, DH), _F32),
               pltpu.VMEM((RPT, CW), _F32),
               pltpu.VMEM_SHARED((NP, DH), _F32),
               pltpu.VMEM_SHARED((NP, CW), _F32)]
            + [pltpu.SemaphoreType.DMA] * (NBUF + 3)
        )
    else:
        out_type = jax.ShapeDtypeStruct((NP, D), _F32)
        scratch = (
            [pltpu.VMEM((2, IDXB, CHUNK), jnp.int32),
             pltpu.VMEM((2, IDXB, CHUNK), jnp.int32)]
            + [pltpu.VMEM((CHUNK, DH), _F32)] * NBUF
            + [pltpu.VMEM((RPT, DH), _F32),
               pltpu.VMEM_SHARED((NP, DH), _F32)]
            + [pltpu.SemaphoreType.DMA] * (NBUF + 2)
        )
    mesh = plsc.VectorSubcoreMesh(core_axis_name="c", subcore_axis_name="s",
                                  num_cores=NC, num_subcores=NS)
    return pl.kernel(
        functools.partial(_sc_agg_body, with_counts),
        out_type=out_type,
        mesh=mesh,
        scratch_types=scratch,
        compiler_params=pltpu.CompilerParams(use_tc_tiling_on_sc=False),
        name="sc_edge_agg" + ("_cnt" if with_counts else ""),
    )


_SC_CACHE = {}


def _sc_agg_call(with_counts, *argv):
    if with_counts not in _SC_CACHE:
        _SC_CACHE[with_counts] = _make_sc_agg(with_counts)
    return _SC_CACHE[with_counts](*argv)


_DOT = dict(dimension_numbers=(((1,), (1,)), ((), ())),
            preferred_element_type=_F32,
            precision=lax.Precision.HIGHEST)

_BLK = 1000
_GRID = N // _BLK


def _tc_pre_body(x_ref, wl_ref, wr_ref, b_ref, y_ref, z_ref):
    xb = x_ref[...]
    y_ref[...] = lax.dot_general(xb, wl_ref[...], **_DOT)
    z_ref[...] = lax.dot_general(xb, wr_ref[...], **_DOT) + b_ref[...]


def _tc_pre(x, W_l, W_r, b):
    return pl.pallas_call(
        _tc_pre_body,
        grid=(_GRID,),
        in_specs=[
            pl.BlockSpec((_BLK, D), lambda i: (i, 0)),
            pl.BlockSpec((D, D), lambda i: (0, 0)),
            pl.BlockSpec((D, D), lambda i: (0, 0)),
            pl.BlockSpec((1, D), lambda i: (0, 0)),
        ],
        out_specs=[
            pl.BlockSpec((_BLK, D), lambda i: (i, 0)),
            pl.BlockSpec((_BLK, D), lambda i: (i, 0)),
        ],
        out_shape=[
            jax.ShapeDtypeStruct((N, D), _F32),
            jax.ShapeDtypeStruct((N, D), _F32),
        ],
        name="tc_pre",
    )(x, W_l, W_r, b.reshape(1, D))


def _mean_relu(s_ref, c_ref, z_ref):
    cnt = c_ref[0, :, 0:1] + c_ref[1, :, 0:1]
    return jnp.maximum(s_ref[...] / jnp.maximum(cnt, 1.0) + z_ref[...], 0.0)


def _tc_mid_body(s_ref, c_ref, z_ref, wl_ref, wr_ref, b_ref, y_ref, z2_ref):
    h = _mean_relu(s_ref, c_ref, z_ref)
    y_ref[...] = lax.dot_general(h, wl_ref[...], **_DOT)
    z2_ref[...] = lax.dot_general(h, wr_ref[...], **_DOT) + b_ref[...]


def _tc_mid(s, c, z, W_l, W_r, b):
    return pl.pallas_call(
        _tc_mid_body,
        grid=(_GRID,),
        in_specs=[
            pl.BlockSpec((_BLK, D), lambda i: (i, 0)),
            pl.BlockSpec((NC, _BLK, CW), lambda i: (0, i, 0)),
            pl.BlockSpec((_BLK, D), lambda i: (i, 0)),
            pl.BlockSpec((D, D), lambda i: (0, 0)),
            pl.BlockSpec((D, D), lambda i: (0, 0)),
            pl.BlockSpec((1, D), lambda i: (0, 0)),
        ],
        out_specs=[
            pl.BlockSpec((_BLK, D), lambda i: (i, 0)),
            pl.BlockSpec((_BLK, D), lambda i: (i, 0)),
        ],
        out_shape=[
            jax.ShapeDtypeStruct((N, D), _F32),
            jax.ShapeDtypeStruct((N, D), _F32),
        ],
        name="tc_mid",
    )(s, c, z, W_l, W_r, b.reshape(1, D))


def _tc_post_body(s_ref, c_ref, z_ref, o_ref):
    o_ref[...] = _mean_relu(s_ref, c_ref, z_ref)


def _tc_post(s, c, z):
    return pl.pallas_call(
        _tc_post_body,
        grid=(_GRID,),
        in_specs=[
            pl.BlockSpec((_BLK, D), lambda i: (i, 0)),
            pl.BlockSpec((NC, _BLK, CW), lambda i: (0, i, 0)),
            pl.BlockSpec((_BLK, D), lambda i: (i, 0)),
        ],
        out_specs=pl.BlockSpec((_BLK, D), lambda i: (i, 0)),
        out_shape=jax.ShapeDtypeStruct((N, D), _F32),
        name="tc_post",
    )(s, c, z)


def kernel(x, edge_index, W1_l, W1_r, b1, W2_l, W2_r, b2):
    e2 = edge_index.reshape(2, CROWS, CHUNK)
    y1, z1 = _tc_pre(x, W1_l, W1_r, b1)
    s1, cnt = _sc_agg_call(True, y1.reshape(2 * N, DH), e2)
    y2, z2 = _tc_mid(s1, cnt, z1, W2_l, W2_r, b2)
    s2 = _sc_agg_call(False, y2.reshape(2 * N, DH), e2)
    return _tc_post(s2, cnt, z2)


# speedup vs baseline: 1.0504x; 1.0504x over previous
import functools

import jax
import jax.numpy as jnp
from jax import lax
from jax.experimental import pallas as pl
from jax.experimental.pallas import tpu as pltpu
from jax.experimental.pallas import tpu_sc as plsc

N = 10000
D = 128
DH = D // 2
E = 320000

NC = 2
NS = 16
EPS = E // NS
CHUNK = 80
NCHUNKS = EPS // CHUNK
IDXB = 25
NBLK = NCHUNKS // IDXB
CROWS = E // CHUNK
CW = 8
NP = 10240
RPT = NP // NS

_F32 = jnp.float32


NBUF = 5


def _sc_agg_body(with_counts, y_hbm, e2_hbm, *refs):
    if with_counts:
        (out_s, out_c, src_i, dst_i, r0, r1, r2, r3, r4, ones_v, zbuf, zcnt,
         agg_sh, cnt_sh, b0, b1, b2_, b3, b4,
         isem0, isem1, csem) = refs
    else:
        (out_s, src_i, dst_i, r0, r1, r2, r3, r4, zbuf,
         agg_sh, b0, b1, b2_, b3, b4,
         isem0, isem1) = refs
    rows = (r0, r1, r2, r3, r4)
    bsem = (b0, b1, b2_, b3, b4)
    isem = (isem0, isem1)

    cid = lax.axis_index("c")
    sid = lax.axis_index("s")

    base = sid * RPT

    crow = sid * NCHUNKS

    def _fire_idx(r, p):
        row = crow + r * IDXB
        pltpu.async_copy(e2_hbm.at[0, pl.ds(row, IDXB)], src_i.at[p], isem[p])
        pltpu.async_copy(e2_hbm.at[1, pl.ds(row, IDXB)], dst_i.at[p], isem[p])

    def _wait_idx(p):
        pltpu.make_async_copy(e2_hbm.at[0, pl.ds(0, IDXB)], src_i.at[p],
                              isem[p]).wait()
        pltpu.make_async_copy(e2_hbm.at[0, pl.ds(0, IDXB)], dst_i.at[p],
                              isem[p]).wait()

    def _add_off(p):
        for c in range(IDXB):
            for k in range(CHUNK // 16):
                sl = pl.ds(k * 16, 16)
                src_i[p, c, sl] = src_i[p, c, sl] * 2 + cid

    def _fire_gather(p, u, b):
        pltpu.async_copy(y_hbm.at[src_i.at[p, u]], rows[b], bsem[b])

    def _wait_gather(b):
        pltpu.make_async_copy(y_hbm.at[pl.ds(0, CHUNK)], rows[b],
                              bsem[b]).wait()

    def _fire_scat(p, u, b):
        pltpu.async_copy(rows[b], agg_sh.at[dst_i.at[p, u]], bsem[b],
                         add=True)

    _wait_scat = _wait_gather

    def _fire_cnt(p, u):
        pltpu.async_copy(ones_v, cnt_sh.at[dst_i.at[p, u]], csem, add=True)

    def _wait_cnt():
        pltpu.make_async_copy(y_hbm.at[pl.ds(0, (CHUNK * CW) // DH)], ones_v,
                              csem).wait()

    _fire_idx(0, 0)
    _fire_idx(1, 1)

    def _zrow(i, _):
        for c in range(DH // 16):
            zbuf[i, pl.ds(c * 16, 16)] = jnp.zeros((16,), _F32)
        if with_counts:
            zcnt[i, pl.ds(0, 16)] = jnp.zeros((16,), _F32)
        return 0

    lax.fori_loop(0, RPT, _zrow, 0)

    if with_counts:
        def _orow(i, _):
            ones_v[i, pl.ds(0, 16)] = jnp.ones((16,), _F32)
            return 0
        lax.fori_loop(0, CHUNK, _orow, 0)

    pltpu.sync_copy(zbuf, agg_sh.at[pl.ds(base, RPT)])
    if with_counts:
        pltpu.sync_copy(zcnt, cnt_sh.at[pl.ds(base, RPT)])

    _wait_idx(0)
    _add_off(0)
    _fire_gather(0, 0, 0)
    _fire_gather(0, 1, 1)

    plsc.subcore_barrier()

    def _outer(t, _):
        for p in range(2):
            for u in range(IDXB):
                b = u % NBUF
                _wait_gather(b)
                _fire_scat(p, u, b)
                if with_counts:
                    @pl.when(cid == (u % 2))
                    def _():
                        if p == 0 and u < 2:
                            @pl.when(t > 0)
                            def _():
                                _wait_cnt()
                        else:
                            _wait_cnt()
                        _fire_cnt(p, u)
                un = u + 2
                if un < IDXB:
                    b2 = un % NBUF
                    if p == 0 and u < 3:
                        @pl.when(t > 0)
                        def _():
                            _wait_scat(b2)
                    else:
                        _wait_scat(b2)
                    _fire_gather(p, un, b2)
                    if u == 2:
                        if p == 0:
                            @pl.when(t > 0)
                            def _():
                                _fire_idx(2 * t + 1, 1)
                        else:
                            @pl.when(t < NBLK // 2 - 1)
                            def _():
                                _fire_idx(2 * t + 2, 0)
                else:
                    uo = un - IDXB
                    if p == 0:
                        if u == IDXB - 2:
                            _wait_idx(1)
                            _add_off(1)
                        _wait_scat(uo)
                        _fire_gather(1, uo, uo)
                    else:
                        @pl.when(t < NBLK // 2 - 1)
                        def _():
                            if u == IDXB - 2:
                                _wait_idx(0)
                                _add_off(0)
                            _wait_scat(uo)
                            _fire_gather(0, uo, uo)
        return 0

    lax.fori_loop(0, NBLK // 2, _outer, 0)

    for b in range(NBUF):
        _wait_scat(b)
    if with_counts:
        _wait_cnt()

    plsc.subcore_barrier()

    pltpu.sync_copy(agg_sh.at[pl.ds(base, RPT)],
                    out_s.at[pl.ds(base, RPT), pl.ds(cid * DH, DH)])
    if with_counts:
        pltpu.sync_copy(cnt_sh.at[pl.ds(base, RPT)],
                        out_c.at[cid, pl.ds(base, RPT)])


def _make_sc_agg(with_counts):
    if with_counts:
        out_type = (jax.ShapeDtypeStruct((NP, D), _F32),
                    jax.ShapeDtypeStruct((NC, NP, CW), _F32))
        scratch = (
            [pltpu.VMEM((2, IDXB, CHUNK), jnp.int32),
             pltpu.VMEM((2, IDXB, CHUNK), jnp.int32)]
            + [pltpu.VMEM((CHUNK, DH), _F32)] * NBUF
            + [pltpu.VMEM((CHUNK, CW), _F32),
               pltpu.VMEM((RPT, DH), _F32),
               pltpu.VMEM((RPT, CW), _F32),
               pltpu.VMEM_SHARED((NP, DH), _F32),
               pltpu.VMEM_SHARED((NP, CW), _F32)]
            + [pltpu.SemaphoreType.DMA] * (NBUF + 3)
        )
    else:
        out_type = jax.ShapeDtypeStruct((NP, D), _F32)
        scratch = (
            [pltpu.VMEM((2, IDXB, CHUNK), jnp.int32),
             pltpu.VMEM((2, IDXB, CHUNK), jnp.int32)]
            + [pltpu.VMEM((CHUNK, DH), _F32)] * NBUF
            + [pltpu.VMEM((RPT, DH), _F32),
               pltpu.VMEM_SHARED((NP, DH), _F32)]
            + [pltpu.SemaphoreType.DMA] * (NBUF + 2)
        )
    mesh = plsc.VectorSubcoreMesh(core_axis_name="c", subcore_axis_name="s",
                                  num_cores=NC, num_subcores=NS)
    return pl.kernel(
        functools.partial(_sc_agg_body, with_counts),
        out_type=out_type,
        mesh=mesh,
        scratch_types=scratch,
        compiler_params=pltpu.CompilerParams(use_tc_tiling_on_sc=False),
        name="sc_edge_agg" + ("_cnt" if with_counts else ""),
    )


_SC_CACHE = {}


def _sc_agg_call(with_counts, *argv):
    if with_counts not in _SC_CACHE:
        _SC_CACHE[with_counts] = _make_sc_agg(with_counts)
    return _SC_CACHE[with_counts](*argv)


_DOT = dict(dimension_numbers=(((1,), (1,)), ((), ())),
            preferred_element_type=_F32)

_BLK = 1000
_GRID = N // _BLK


def _tc_pre_body(x_ref, wl_ref, wr_ref, b_ref, y_ref, z_ref):
    xb = x_ref[...]
    y_ref[...] = lax.dot_general(xb, wl_ref[...], **_DOT)
    z_ref[...] = lax.dot_general(xb, wr_ref[...], **_DOT) + b_ref[...]


def _tc_pre(x, W_l, W_r, b):
    return pl.pallas_call(
        _tc_pre_body,
        grid=(_GRID,),
        in_specs=[
            pl.BlockSpec((_BLK, D), lambda i: (i, 0)),
            pl.BlockSpec((D, D), lambda i: (0, 0)),
            pl.BlockSpec((D, D), lambda i: (0, 0)),
            pl.BlockSpec((1, D), lambda i: (0, 0)),
        ],
        out_specs=[
            pl.BlockSpec((_BLK, D), lambda i: (i, 0)),
            pl.BlockSpec((_BLK, D), lambda i: (i, 0)),
        ],
        out_shape=[
            jax.ShapeDtypeStruct((N, D), _F32),
            jax.ShapeDtypeStruct((N, D), _F32),
        ],
        name="tc_pre",
    )(x, W_l, W_r, b.reshape(1, D))


def _mean_relu(s_ref, c_ref, z_ref):
    cnt = c_ref[0, :, 0:1] + c_ref[1, :, 0:1]
    return jnp.maximum(s_ref[...] / jnp.maximum(cnt, 1.0) + z_ref[...], 0.0)


def _tc_mid_body(s_ref, c_ref, z_ref, wl_ref, wr_ref, b_ref, y_ref, z2_ref):
    h = _mean_relu(s_ref, c_ref, z_ref)
    y_ref[...] = lax.dot_general(h, wl_ref[...], **_DOT)
    z2_ref[...] = lax.dot_general(h, wr_ref[...], **_DOT) + b_ref[...]


def _tc_mid(s, c, z, W_l, W_r, b):
    return pl.pallas_call(
        _tc_mid_body,
        grid=(_GRID,),
        in_specs=[
            pl.BlockSpec((_BLK, D), lambda i: (i, 0)),
            pl.BlockSpec((NC, _BLK, CW), lambda i: (0, i, 0)),
            pl.BlockSpec((_BLK, D), lambda i: (i, 0)),
            pl.BlockSpec((D, D), lambda i: (0, 0)),
            pl.BlockSpec((D, D), lambda i: (0, 0)),
            pl.BlockSpec((1, D), lambda i: (0, 0)),
        ],
        out_specs=[
            pl.BlockSpec((_BLK, D), lambda i: (i, 0)),
            pl.BlockSpec((_BLK, D), lambda i: (i, 0)),
        ],
        out_shape=[
            jax.ShapeDtypeStruct((N, D), _F32),
            jax.ShapeDtypeStruct((N, D), _F32),
        ],
        name="tc_mid",
    )(s, c, z, W_l, W_r, b.reshape(1, D))


def _tc_post_body(s_ref, c_ref, z_ref, o_ref):
    o_ref[...] = _mean_relu(s_ref, c_ref, z_ref)


def _tc_post(s, c, z):
    return pl.pallas_call(
        _tc_post_body,
        grid=(_GRID,),
        in_specs=[
            pl.BlockSpec((_BLK, D), lambda i: (i, 0)),
            pl.BlockSpec((NC, _BLK, CW), lambda i: (0, i, 0)),
            pl.BlockSpec((_BLK, D), lambda i: (i, 0)),
        ],
        out_specs=pl.BlockSpec((_BLK, D), lambda i: (i, 0)),
        out_shape=jax.ShapeDtypeStruct((N, D), _F32),
        name="tc_post",
    )(s, c, z)


def kernel(x, edge_index, W1_l, W1_r, b1, W2_l, W2_r, b2):
    e2 = edge_index.reshape(2, CROWS, CHUNK)
    y1, z1 = _tc_pre(x, W1_l, W1_r, b1)
    s1, cnt = _sc_agg_call(True, y1.reshape(2 * N, DH), e2)
    y2, z2 = _tc_mid(s1, cnt, z1, W2_l, W2_r, b2)
    s2 = _sc_agg_call(False, y2.reshape(2 * N, DH), e2)
    return _tc_post(s2, cnt, z2)


# speedup vs baseline: 1.0745x; 1.0230x over previous
import functools

import jax
import jax.numpy as jnp
from jax import lax
from jax.experimental import pallas as pl
from jax.experimental.pallas import tpu as pltpu
from jax.experimental.pallas import tpu_sc as plsc

N = 10000
D = 128
DH = D // 2
E = 320000

NC = 2
NS = 16
EPS = E // NS
CHUNK = 80
NCHUNKS = EPS // CHUNK
IDXB = 25
NBLK = NCHUNKS // IDXB
CROWS = E // CHUNK
CW = 8
NP = 10240
RPT = NP // NS

_F32 = jnp.float32


NBUF = 5


def _sc_agg_body(with_counts, y_hbm, e2_hbm, *refs):
    if with_counts:
        (out_s, out_c, src_i, dst_i, r0, r1, r2, r3, r4, ones_v, zbuf, zcnt,
         agg_sh, cnt_sh, b0, b1, b2_, b3, b4,
         isem0, isem1, csem) = refs
    else:
        (out_s, src_i, dst_i, r0, r1, r2, r3, r4, zbuf,
         agg_sh, b0, b1, b2_, b3, b4,
         isem0, isem1) = refs
    rows = (r0, r1, r2, r3, r4)
    bsem = (b0, b1, b2_, b3, b4)
    isem = (isem0, isem1)

    cid = lax.axis_index("c")
    sid = lax.axis_index("s")

    base = sid * RPT

    crow = sid * NCHUNKS

    def _fire_idx(r, p):
        row = crow + r * IDXB
        pltpu.async_copy(e2_hbm.at[0, pl.ds(row, IDXB)], src_i.at[p], isem[p])
        pltpu.async_copy(e2_hbm.at[1, pl.ds(row, IDXB)], dst_i.at[p], isem[p])

    def _wait_idx(p):
        pltpu.make_async_copy(e2_hbm.at[0, pl.ds(0, IDXB)], src_i.at[p],
                              isem[p]).wait()
        pltpu.make_async_copy(e2_hbm.at[0, pl.ds(0, IDXB)], dst_i.at[p],
                              isem[p]).wait()

    def _add_off(p):
        for c in range(IDXB):
            for k in range(CHUNK // 16):
                sl = pl.ds(k * 16, 16)
                src_i[p, c, sl] = src_i[p, c, sl] * 2 + cid

    def _fire_gather(p, u, b):
        pltpu.async_copy(y_hbm.at[src_i.at[p, u]], rows[b], bsem[b])

    def _wait_gather(b):
        pltpu.make_async_copy(y_hbm.at[pl.ds(0, CHUNK)], rows[b],
                              bsem[b]).wait()

    def _fire_scat(p, u, b):
        pltpu.async_copy(rows[b], agg_sh.at[dst_i.at[p, u]], bsem[b],
                         add=True)

    _wait_scat = _wait_gather

    def _fire_cnt(p, u):
        pltpu.async_copy(ones_v, cnt_sh.at[dst_i.at[p, u]], csem, add=True)

    def _wait_cnt():
        pltpu.make_async_copy(y_hbm.at[pl.ds(0, (CHUNK * CW) // DH)], ones_v,
                              csem).wait()

    _fire_idx(0, 0)
    _fire_idx(1, 1)

    def _zrow(i, _):
        for c in range(DH // 16):
            zbuf[i, pl.ds(c * 16, 16)] = jnp.zeros((16,), _F32)
        if with_counts:
            zcnt[i, pl.ds(0, 16)] = jnp.zeros((16,), _F32)
        return 0

    lax.fori_loop(0, RPT, _zrow, 0)

    if with_counts:
        def _orow(i, _):
            ones_v[i, pl.ds(0, 16)] = jnp.ones((16,), _F32)
            return 0
        lax.fori_loop(0, CHUNK, _orow, 0)

    pltpu.sync_copy(zbuf, agg_sh.at[pl.ds(base, RPT)])
    if with_counts:
        pltpu.sync_copy(zcnt, cnt_sh.at[pl.ds(base, RPT)])

    _wait_idx(0)
    _add_off(0)
    _fire_gather(0, 0, 0)
    _fire_gather(0, 1, 1)

    plsc.subcore_barrier()

    def _outer(t, _):
        for p in range(2):
            for u in range(IDXB):
                b = u % NBUF
                _wait_gather(b)
                _fire_scat(p, u, b)
                if with_counts:
                    @pl.when(cid == (u % 2))
                    def _():
                        if p == 0 and u < 2:
                            @pl.when(t > 0)
                            def _():
                                _wait_cnt()
                        else:
                            _wait_cnt()
                        _fire_cnt(p, u)
                un = u + 2
                if un < IDXB:
                    b2 = un % NBUF
                    if p == 0 and u < 3:
                        @pl.when(t > 0)
                        def _():
                            _wait_scat(b2)
                    else:
                        _wait_scat(b2)
                    _fire_gather(p, un, b2)
                    if u == 2:
                        if p == 0:
                            @pl.when(t > 0)
                            def _():
                                _fire_idx(2 * t + 1, 1)
                        else:
                            @pl.when(t < NBLK // 2 - 1)
                            def _():
                                _fire_idx(2 * t + 2, 0)
                else:
                    uo = un - IDXB
                    if p == 0:
                        if u == IDXB - 2:
                            _wait_idx(1)
                            _add_off(1)
                        _wait_scat(uo)
                        _fire_gather(1, uo, uo)
                    else:
                        @pl.when(t < NBLK // 2 - 1)
                        def _():
                            if u == IDXB - 2:
                                _wait_idx(0)
                                _add_off(0)
                            _wait_scat(uo)
                            _fire_gather(0, uo, uo)
        return 0

    lax.fori_loop(0, NBLK // 2, _outer, 0)

    for b in range(NBUF):
        _wait_scat(b)
    if with_counts:
        _wait_cnt()

    plsc.subcore_barrier()

    pltpu.sync_copy(agg_sh.at[pl.ds(base, RPT)],
                    out_s.at[pl.ds(base, RPT), pl.ds(cid * DH, DH)])
    if with_counts:
        pltpu.sync_copy(cnt_sh.at[pl.ds(base, RPT)],
                        out_c.at[cid, pl.ds(base, RPT)])


def _make_sc_agg(with_counts):
    if with_counts:
        out_type = (jax.ShapeDtypeStruct((NP, D), _F32),
                    jax.ShapeDtypeStruct((NC, NP, CW), _F32))
        scratch = (
            [pltpu.VMEM((2, IDXB, CHUNK), jnp.int32),
             pltpu.VMEM((2, IDXB, CHUNK), jnp.int32)]
            + [pltpu.VMEM((CHUNK, DH), _F32)] * NBUF
            + [pltpu.VMEM((CHUNK, CW), _F32),
               pltpu.VMEM((RPT, DH), _F32),
               pltpu.VMEM((RPT, CW), _F32),
               pltpu.VMEM_SHARED((NP, DH), _F32),
               pltpu.VMEM_SHARED((NP, CW), _F32)]
            + [pltpu.SemaphoreType.DMA] * (NBUF + 3)
        )
    else:
        out_type = jax.ShapeDtypeStruct((NP, D), _F32)
        scratch = (
            [pltpu.VMEM((2, IDXB, CHUNK), jnp.int32),
             pltpu.VMEM((2, IDXB, CHUNK), jnp.int32)]
            + [pltpu.VMEM((CHUNK, DH), _F32)] * NBUF
            + [pltpu.VMEM((RPT, DH), _F32),
               pltpu.VMEM_SHARED((NP, DH), _F32)]
            + [pltpu.SemaphoreType.DMA] * (NBUF + 2)
        )
    mesh = plsc.VectorSubcoreMesh(core_axis_name="c", subcore_axis_name="s",
                                  num_cores=NC, num_subcores=NS)
    return pl.kernel(
        functools.partial(_sc_agg_body, with_counts),
        out_type=out_type,
        mesh=mesh,
        scratch_types=scratch,
        compiler_params=pltpu.CompilerParams(use_tc_tiling_on_sc=False),
        name="sc_edge_agg" + ("_cnt" if with_counts else ""),
    )


_SC_CACHE = {}


def _sc_agg_call(with_counts, *argv):
    if with_counts not in _SC_CACHE:
        _SC_CACHE[with_counts] = _make_sc_agg(with_counts)
    return _SC_CACHE[with_counts](*argv)


_DOT = dict(dimension_numbers=(((1,), (1,)), ((), ())),
            preferred_element_type=_F32)

_BLK = 2000
_GRID = N // _BLK


def _tc_pre_body(x_ref, wl_ref, wr_ref, b_ref, y_ref, z_ref):
    xb = x_ref[...]
    y_ref[...] = lax.dot_general(xb, wl_ref[...], **_DOT)
    z_ref[...] = lax.dot_general(xb, wr_ref[...], **_DOT) + b_ref[...]


def _tc_pre(x, W_l, W_r, b):
    return pl.pallas_call(
        _tc_pre_body,
        grid=(_GRID,),
        in_specs=[
            pl.BlockSpec((_BLK, D), lambda i: (i, 0)),
            pl.BlockSpec((D, D), lambda i: (0, 0)),
            pl.BlockSpec((D, D), lambda i: (0, 0)),
            pl.BlockSpec((1, D), lambda i: (0, 0)),
        ],
        out_specs=[
            pl.BlockSpec((_BLK, D), lambda i: (i, 0)),
            pl.BlockSpec((_BLK, D), lambda i: (i, 0)),
        ],
        out_shape=[
            jax.ShapeDtypeStruct((N, D), _F32),
            jax.ShapeDtypeStruct((N, D), _F32),
        ],
        name="tc_pre",
    )(x, W_l, W_r, b.reshape(1, D))


def _mean_relu(s_ref, c_ref, z_ref):
    cnt = c_ref[0, :, 0:1] + c_ref[1, :, 0:1]
    return jnp.maximum(s_ref[...] / jnp.maximum(cnt, 1.0) + z_ref[...], 0.0)


def _tc_mid_body(s_ref, c_ref, z_ref, wl_ref, wr_ref, b_ref, y_ref, z2_ref):
    h = _mean_relu(s_ref, c_ref, z_ref)
    y_ref[...] = lax.dot_general(h, wl_ref[...], **_DOT)
    z2_ref[...] = lax.dot_general(h, wr_ref[...], **_DOT) + b_ref[...]


def _tc_mid(s, c, z, W_l, W_r, b):
    return pl.pallas_call(
        _tc_mid_body,
        grid=(_GRID,),
        in_specs=[
            pl.BlockSpec((_BLK, D), lambda i: (i, 0)),
            pl.BlockSpec((NC, _BLK, CW), lambda i: (0, i, 0)),
            pl.BlockSpec((_BLK, D), lambda i: (i, 0)),
            pl.BlockSpec((D, D), lambda i: (0, 0)),
            pl.BlockSpec((D, D), lambda i: (0, 0)),
            pl.BlockSpec((1, D), lambda i: (0, 0)),
        ],
        out_specs=[
            pl.BlockSpec((_BLK, D), lambda i: (i, 0)),
            pl.BlockSpec((_BLK, D), lambda i: (i, 0)),
        ],
        out_shape=[
            jax.ShapeDtypeStruct((N, D), _F32),
            jax.ShapeDtypeStruct((N, D), _F32),
        ],
        name="tc_mid",
    )(s, c, z, W_l, W_r, b.reshape(1, D))


def _tc_post_body(s_ref, c_ref, z_ref, o_ref):
    o_ref[...] = _mean_relu(s_ref, c_ref, z_ref)


def _tc_post(s, c, z):
    return pl.pallas_call(
        _tc_post_body,
        grid=(_GRID,),
        in_specs=[
            pl.BlockSpec((_BLK, D), lambda i: (i, 0)),
            pl.BlockSpec((NC, _BLK, CW), lambda i: (0, i, 0)),
            pl.BlockSpec((_BLK, D), lambda i: (i, 0)),
        ],
        out_specs=pl.BlockSpec((_BLK, D), lambda i: (i, 0)),
        out_shape=jax.ShapeDtypeStruct((N, D), _F32),
        name="tc_post",
    )(s, c, z)


def kernel(x, edge_index, W1_l, W1_r, b1, W2_l, W2_r, b2):
    e2 = edge_index.reshape(2, CROWS, CHUNK)
    y1, z1 = _tc_pre(x, W1_l, W1_r, b1)
    s1, cnt = _sc_agg_call(True, y1.reshape(2 * N, DH), e2)
    y2, z2 = _tc_mid(s1, cnt, z1, W2_l, W2_r, b2)
    s2 = _sc_agg_call(False, y2.reshape(2 * N, DH), e2)
    return _tc_post(s2, cnt, z2)


# speedup vs baseline: 1.0858x; 1.0105x over previous
import functools

import jax
import jax.numpy as jnp
from jax import lax
from jax.experimental import pallas as pl
from jax.experimental.pallas import tpu as pltpu
from jax.experimental.pallas import tpu_sc as plsc

N = 10000
D = 128
DH = D // 2
E = 320000

NC = 2
NS = 16
EPS = E // NS
CHUNK = 80
NCHUNKS = EPS // CHUNK
IDXB = 25
NBLK = NCHUNKS // IDXB
CROWS = E // CHUNK
CW = 8
NP = 10240
RPT = NP // NS

_F32 = jnp.float32


NBUF = 5


def _sc_agg_body(with_counts, y_hbm, e2_hbm, *refs):
    if with_counts:
        (out_s, out_c, src_i, dst_i, r0, r1, r2, r3, r4, ones_v, zbuf, zcnt,
         agg_sh, cnt_sh, b0, b1, b2_, b3, b4,
         isem0, isem1, csem) = refs
    else:
        (out_s, src_i, dst_i, r0, r1, r2, r3, r4, zbuf,
         agg_sh, b0, b1, b2_, b3, b4,
         isem0, isem1) = refs
    rows = (r0, r1, r2, r3, r4)
    bsem = (b0, b1, b2_, b3, b4)
    isem = (isem0, isem1)

    cid = lax.axis_index("c")
    sid = lax.axis_index("s")

    base = sid * RPT

    crow = sid * NCHUNKS

    def _fire_idx(r, p):
        row = crow + r * IDXB
        pltpu.async_copy(e2_hbm.at[0, pl.ds(row, IDXB)], src_i.at[p], isem[p])
        pltpu.async_copy(e2_hbm.at[1, pl.ds(row, IDXB)], dst_i.at[p], isem[p])

    def _wait_idx(p):
        pltpu.make_async_copy(e2_hbm.at[0, pl.ds(0, IDXB)], src_i.at[p],
                              isem[p]).wait()
        pltpu.make_async_copy(e2_hbm.at[0, pl.ds(0, IDXB)], dst_i.at[p],
                              isem[p]).wait()

    def _add_off(p):
        for c in range(IDXB):
            for k in range(CHUNK // 16):
                sl = pl.ds(k * 16, 16)
                src_i[p, c, sl] = src_i[p, c, sl] * 2 + cid

    def _fire_gather(p, u, b):
        pltpu.async_copy(y_hbm.at[src_i.at[p, u]], rows[b], bsem[b])

    def _wait_gather(b):
        pltpu.make_async_copy(y_hbm.at[pl.ds(0, CHUNK)], rows[b],
                              bsem[b]).wait()

    def _fire_scat(p, u, b):
        pltpu.async_copy(rows[b], agg_sh.at[dst_i.at[p, u]], bsem[b],
                         add=True)

    _wait_scat = _wait_gather

    def _fire_cnt(p, u):
        pltpu.async_copy(ones_v, cnt_sh.at[dst_i.at[p, u]], csem, add=True)

    def _wait_cnt():
        pltpu.make_async_copy(y_hbm.at[pl.ds(0, (CHUNK * CW) // DH)], ones_v,
                              csem).wait()

    _fire_idx(0, 0)
    _fire_idx(1, 1)

    def _zrow(i, _):
        for c in range(DH // 16):
            zbuf[i, pl.ds(c * 16, 16)] = jnp.zeros((16,), _F32)
        if with_counts:
            zcnt[i, pl.ds(0, 16)] = jnp.zeros((16,), _F32)
        return 0

    lax.fori_loop(0, RPT, _zrow, 0)

    if with_counts:
        def _orow(i, _):
            ones_v[i, pl.ds(0, 16)] = jnp.ones((16,), _F32)
            return 0
        lax.fori_loop(0, CHUNK, _orow, 0)

    pltpu.sync_copy(zbuf, agg_sh.at[pl.ds(base, RPT)])
    if with_counts:
        pltpu.sync_copy(zcnt, cnt_sh.at[pl.ds(base, RPT)])

    _wait_idx(0)
    _add_off(0)
    _fire_gather(0, 0, 0)
    _fire_gather(0, 1, 1)

    plsc.subcore_barrier()

    def _outer(t, _):
        for p in range(2):
            for u in range(IDXB):
                b = u % NBUF
                _wait_gather(b)
                _fire_scat(p, u, b)
                if with_counts:
                    @pl.when(cid == (u % 2))
                    def _():
                        if p == 0 and u < 2:
                            @pl.when(t > 0)
                            def _():
                                _wait_cnt()
                        else:
                            _wait_cnt()
                        _fire_cnt(p, u)
                un = u + 2
                if un < IDXB:
                    b2 = un % NBUF
                    if p == 0 and u < 3:
                        @pl.when(t > 0)
                        def _():
                            _wait_scat(b2)
                    else:
                        _wait_scat(b2)
                    _fire_gather(p, un, b2)
                    if u == 2:
                        if p == 0:
                            @pl.when(t > 0)
                            def _():
                                _fire_idx(2 * t + 1, 1)
                        else:
                            @pl.when(t < NBLK // 2 - 1)
                            def _():
                                _fire_idx(2 * t + 2, 0)
                else:
                    uo = un - IDXB
                    if p == 0:
                        if u == IDXB - 2:
                            _wait_idx(1)
                            _add_off(1)
                        _wait_scat(uo)
                        _fire_gather(1, uo, uo)
                    else:
                        @pl.when(t < NBLK // 2 - 1)
                        def _():
                            if u == IDXB - 2:
                                _wait_idx(0)
                                _add_off(0)
                            _wait_scat(uo)
                            _fire_gather(0, uo, uo)
        return 0

    lax.fori_loop(0, NBLK // 2, _outer, 0)

    for b in range(NBUF):
        _wait_scat(b)
    if with_counts:
        _wait_cnt()

    plsc.subcore_barrier()

    pltpu.sync_copy(agg_sh.at[pl.ds(base, RPT)],
                    out_s.at[pl.ds(base, RPT), pl.ds(cid * DH, DH)])
    if with_counts:
        pltpu.sync_copy(cnt_sh.at[pl.ds(base, RPT)],
                        out_c.at[cid, pl.ds(base, RPT)])


def _make_sc_agg(with_counts):
    if with_counts:
        out_type = (jax.ShapeDtypeStruct((NP, D), _F32),
                    jax.ShapeDtypeStruct((NC, NP, CW), _F32))
        scratch = (
            [pltpu.VMEM((2, IDXB, CHUNK), jnp.int32),
             pltpu.VMEM((2, IDXB, CHUNK), jnp.int32)]
            + [pltpu.VMEM((CHUNK, DH), _F32)] * NBUF
            + [pltpu.VMEM((CHUNK, CW), _F32),
               pltpu.VMEM((RPT, DH), _F32),
               pltpu.VMEM((RPT, CW), _F32),
               pltpu.VMEM_SHARED((NP, DH), _F32),
               pltpu.VMEM_SHARED((NP, CW), _F32)]
            + [pltpu.SemaphoreType.DMA] * (NBUF + 3)
        )
    else:
        out_type = jax.ShapeDtypeStruct((NP, D), _F32)
        scratch = (
            [pltpu.VMEM((2, IDXB, CHUNK), jnp.int32),
             pltpu.VMEM((2, IDXB, CHUNK), jnp.int32)]
            + [pltpu.VMEM((CHUNK, DH), _F32)] * NBUF
            + [pltpu.VMEM((RPT, DH), _F32),
               pltpu.VMEM_SHARED((NP, DH), _F32)]
            + [pltpu.SemaphoreType.DMA] * (NBUF + 2)
        )
    mesh = plsc.VectorSubcoreMesh(core_axis_name="c", subcore_axis_name="s",
                                  num_cores=NC, num_subcores=NS)
    return pl.kernel(
        functools.partial(_sc_agg_body, with_counts),
        out_type=out_type,
        mesh=mesh,
        scratch_types=scratch,
        compiler_params=pltpu.CompilerParams(use_tc_tiling_on_sc=False),
        name="sc_edge_agg" + ("_cnt" if with_counts else ""),
    )


_SC_CACHE = {}


def _sc_agg_call(with_counts, *argv):
    if with_counts not in _SC_CACHE:
        _SC_CACHE[with_counts] = _make_sc_agg(with_counts)
    return _SC_CACHE[with_counts](*argv)


_DOT = dict(dimension_numbers=(((1,), (1,)), ((), ())),
            preferred_element_type=_F32)

_BLK = 5000
_GRID = N // _BLK


def _tc_pre_body(x_ref, wl_ref, wr_ref, b_ref, y_ref, z_ref):
    xb = x_ref[...]
    y_ref[...] = lax.dot_general(xb, wl_ref[...], **_DOT)
    z_ref[...] = lax.dot_general(xb, wr_ref[...], **_DOT) + b_ref[...]


def _tc_pre(x, W_l, W_r, b):
    return pl.pallas_call(
        _tc_pre_body,
        grid=(_GRID,),
        in_specs=[
            pl.BlockSpec((_BLK, D), lambda i: (i, 0)),
            pl.BlockSpec((D, D), lambda i: (0, 0)),
            pl.BlockSpec((D, D), lambda i: (0, 0)),
            pl.BlockSpec((1, D), lambda i: (0, 0)),
        ],
        out_specs=[
            pl.BlockSpec((_BLK, D), lambda i: (i, 0)),
            pl.BlockSpec((_BLK, D), lambda i: (i, 0)),
        ],
        out_shape=[
            jax.ShapeDtypeStruct((N, D), _F32),
            jax.ShapeDtypeStruct((N, D), _F32),
        ],
        name="tc_pre",
    )(x, W_l, W_r, b.reshape(1, D))


def _mean_relu(s_ref, c_ref, z_ref):
    cnt = c_ref[0, :, 0:1] + c_ref[1, :, 0:1]
    return jnp.maximum(s_ref[...] / jnp.maximum(cnt, 1.0) + z_ref[...], 0.0)


def _tc_mid_body(s_ref, c_ref, z_ref, wl_ref, wr_ref, b_ref, y_ref, z2_ref):
    h = _mean_relu(s_ref, c_ref, z_ref)
    y_ref[...] = lax.dot_general(h, wl_ref[...], **_DOT)
    z2_ref[...] = lax.dot_general(h, wr_ref[...], **_DOT) + b_ref[...]


def _tc_mid(s, c, z, W_l, W_r, b):
    return pl.pallas_call(
        _tc_mid_body,
        grid=(_GRID,),
        in_specs=[
            pl.BlockSpec((_BLK, D), lambda i: (i, 0)),
            pl.BlockSpec((NC, _BLK, CW), lambda i: (0, i, 0)),
            pl.BlockSpec((_BLK, D), lambda i: (i, 0)),
            pl.BlockSpec((D, D), lambda i: (0, 0)),
            pl.BlockSpec((D, D), lambda i: (0, 0)),
            pl.BlockSpec((1, D), lambda i: (0, 0)),
        ],
        out_specs=[
            pl.BlockSpec((_BLK, D), lambda i: (i, 0)),
            pl.BlockSpec((_BLK, D), lambda i: (i, 0)),
        ],
        out_shape=[
            jax.ShapeDtypeStruct((N, D), _F32),
            jax.ShapeDtypeStruct((N, D), _F32),
        ],
        name="tc_mid",
    )(s, c, z, W_l, W_r, b.reshape(1, D))


def _tc_post_body(s_ref, c_ref, z_ref, o_ref):
    o_ref[...] = _mean_relu(s_ref, c_ref, z_ref)


def _tc_post(s, c, z):
    return pl.pallas_call(
        _tc_post_body,
        grid=(_GRID,),
        in_specs=[
            pl.BlockSpec((_BLK, D), lambda i: (i, 0)),
            pl.BlockSpec((NC, _BLK, CW), lambda i: (0, i, 0)),
            pl.BlockSpec((_BLK, D), lambda i: (i, 0)),
        ],
        out_specs=pl.BlockSpec((_BLK, D), lambda i: (i, 0)),
        out_shape=jax.ShapeDtypeStruct((N, D), _F32),
        name="tc_post",
    )(s, c, z)


def kernel(x, edge_index, W1_l, W1_r, b1, W2_l, W2_r, b2):
    e2 = edge_index.reshape(2, CROWS, CHUNK)
    y1, z1 = _tc_pre(x, W1_l, W1_r, b1)
    s1, cnt = _sc_agg_call(True, y1.reshape(2 * N, DH), e2)
    y2, z2 = _tc_mid(s1, cnt, z1, W2_l, W2_r, b2)
    s2 = _sc_agg_call(False, y2.reshape(2 * N, DH), e2)
    return _tc_post(s2, cnt, z2)
